# Initial kernel scaffold; baseline (speedup 1.0000x reference)
import jax
import jax.numpy as jnp
from jax.experimental import pallas as pl


def kernel(x, table):
    raise NotImplementedError("write your pallas kernel here")


# speedup vs baseline: 1.4731x; 1.4731x over previous
import functools
import math

import jax
import jax.numpy as jnp
from jax import lax
from jax.experimental import pallas as pl
from jax.experimental.pallas import tpu as pltpu
from jax.experimental.pallas import tpu_sc as plsc

DIM = 32
LANES = 16
NC = 2
NS = 16
NW = NC * NS
SCALE = math.sqrt(DIM)

CHUNK = 512
NBUF = 2
ROWS_PER_STEP = 8


@functools.lru_cache(maxsize=None)
def _build(batch, hist, vocab):
  B = batch * hist
  assert B % NW == 0
  bpw = B // NW
  assert bpw % CHUNK == 0
  nchunk = bpw // CHUNK

  mesh = plsc.VectorSubcoreMesh(core_axis_name="c", subcore_axis_name="s")

  @functools.partial(
      pl.kernel,
      out_type=jax.ShapeDtypeStruct((B, DIM), jnp.float32),
      mesh=mesh,
      compiler_params=pltpu.CompilerParams(use_tc_tiling_on_sc=False),
      scratch_types=(
          [pltpu.VMEM((bpw,), jnp.int32)]
          + [pltpu.VMEM((CHUNK, DIM), jnp.float32) for _ in range(2 * NBUF)]
          + [pltpu.SemaphoreType.DMA for _ in range(2 * NBUF)]
      ),
  )
  def k(x_hbm, table_hbm, out_hbm, idx_v, g0, g1, o0, o1, gs0, gs1, os0, os1):
    gbuf = (g0, g1)
    obuf = (o0, o1)
    gsem = (gs0, gs1)
    osem = (os0, os1)

    wid = lax.axis_index("s") * NC + lax.axis_index("c")
    base = wid * bpw
    pltpu.sync_copy(x_hbm.at[pl.ds(base, bpw)], idx_v)

    def start_gather(c):
      b = c % NBUF
      pltpu.async_copy(
          table_hbm.at[idx_v.at[pl.ds(c * CHUNK, CHUNK)]], gbuf[b], gsem[b])

    def wait_gather(c):
      b = c % NBUF
      pltpu.make_async_copy(
          table_hbm.at[idx_v.at[pl.ds(c * CHUNK, CHUNK)]], gbuf[b],
          gsem[b]).wait()

    def start_out(c):
      b = c % NBUF
      pltpu.async_copy(
          obuf[b], out_hbm.at[pl.ds(base + c * CHUNK, CHUNK)], osem[b])

    def wait_out(c):
      b = c % NBUF
      pltpu.make_async_copy(
          obuf[b], out_hbm.at[pl.ds(base + c * CHUNK, CHUNK)], osem[b]).wait()

    def scale_chunk(b):
      src = gbuf[b]
      dst = obuf[b]

      def row_block(i, carry):
        r0 = i * ROWS_PER_STEP
        for j in range(ROWS_PER_STEP):
          r = r0 + j
          dst[r, pl.ds(0, LANES)] = src[r, pl.ds(0, LANES)] * SCALE
          dst[r, pl.ds(LANES, LANES)] = src[r, pl.ds(LANES, LANES)] * SCALE
        return carry

      lax.fori_loop(0, CHUNK // ROWS_PER_STEP, row_block, 0)

    for c in range(min(NBUF, nchunk)):
      start_gather(c)
    for c in range(nchunk):
      b = c % NBUF
      wait_gather(c)
      if c >= NBUF:
        wait_out(c - NBUF)
      scale_chunk(b)
      start_out(c)
      if c + NBUF < nchunk:
        start_gather(c + NBUF)
    for c in range(max(0, nchunk - NBUF), nchunk):
      wait_out(c)

  return k


def kernel(x, table):
  batch, hist = x.shape
  vocab, dim = table.shape
  xf = x.reshape(-1).astype(jnp.int32)
  out = _build(batch, hist, vocab)(xf, table)
  return out.reshape(batch, hist, dim)


# speedup vs baseline: 2.0098x; 1.3643x over previous
import functools
import math

import jax
import jax.numpy as jnp
from jax import lax
from jax.experimental import pallas as pl
from jax.experimental.pallas import tpu as pltpu
from jax.experimental.pallas import tpu_sc as plsc

DIM = 32
LANES = 16
NC = 2
NS = 16
NW = NC * NS
SCALE = math.sqrt(DIM)

CHUNK = 512
NBUF = 2
ROWS_PER_STEP = 8


@functools.lru_cache(maxsize=None)
def _build(batch, hist, vocab):
  B = batch * hist
  assert B % NW == 0
  bpw = B // NW
  assert bpw % CHUNK == 0
  nchunk = bpw // CHUNK

  mesh = plsc.VectorSubcoreMesh(core_axis_name="c", subcore_axis_name="s")

  @functools.partial(
      pl.kernel,
      out_type=jax.ShapeDtypeStruct((B, 128), jnp.float32),
      mesh=mesh,
      compiler_params=pltpu.CompilerParams(use_tc_tiling_on_sc=False),
      scratch_types=(
          [pltpu.VMEM((bpw,), jnp.int32)]
          + [pltpu.VMEM((CHUNK, DIM), jnp.float32) for _ in range(2 * NBUF)]
          + [pltpu.SemaphoreType.DMA for _ in range(2 * NBUF)]
      ),
  )
  def k(x_hbm, table_hbm, out_hbm, idx_v, g0, g1, o0, o1, gs0, gs1, os0, os1):
    gbuf = (g0, g1)
    obuf = (o0, o1)
    gsem = (gs0, gs1)
    osem = (os0, os1)

    wid = lax.axis_index("s") * NC + lax.axis_index("c")
    base = wid * bpw
    pltpu.sync_copy(x_hbm.at[pl.ds(base, bpw)], idx_v)

    def start_gather(c):
      b = c % NBUF
      pltpu.async_copy(
          table_hbm.at[idx_v.at[pl.ds(c * CHUNK, CHUNK)]], gbuf[b], gsem[b])

    def wait_gather(c):
      b = c % NBUF
      pltpu.make_async_copy(
          table_hbm.at[idx_v.at[pl.ds(c * CHUNK, CHUNK)]], gbuf[b],
          gsem[b]).wait()

    def start_out(c):
      b = c % NBUF
      pltpu.async_copy(
          obuf[b],
          out_hbm.at[pl.ds(base + c * CHUNK, CHUNK), pl.ds(0, DIM)], osem[b])

    def wait_out(c):
      b = c % NBUF
      pltpu.make_async_copy(
          obuf[b],
          out_hbm.at[pl.ds(base + c * CHUNK, CHUNK), pl.ds(0, DIM)],
          osem[b]).wait()

    def scale_chunk(b):
      src = gbuf[b]
      dst = obuf[b]

      def row_block(i, carry):
        r0 = i * ROWS_PER_STEP
        for j in range(ROWS_PER_STEP):
          r = r0 + j
          dst[r, pl.ds(0, LANES)] = src[r, pl.ds(0, LANES)] * SCALE
          dst[r, pl.ds(LANES, LANES)] = src[r, pl.ds(LANES, LANES)] * SCALE
        return carry

      lax.fori_loop(0, CHUNK // ROWS_PER_STEP, row_block, 0)

    for c in range(min(NBUF, nchunk)):
      start_gather(c)
    for c in range(nchunk):
      b = c % NBUF
      wait_gather(c)
      if c >= NBUF:
        wait_out(c - NBUF)
      scale_chunk(b)
      start_out(c)
      if c + NBUF < nchunk:
        start_gather(c + NBUF)
    for c in range(max(0, nchunk - NBUF), nchunk):
      wait_out(c)

  return k


def kernel(x, table):
  batch, hist = x.shape
  vocab, dim = table.shape
  xf = x.reshape(-1).astype(jnp.int32)
  out = _build(batch, hist, vocab)(xf, table)
  return out[:, :dim].reshape(batch, hist, dim)
